# Initial kernel scaffold; baseline (speedup 1.0000x reference)
import jax
import jax.numpy as jnp
from jax.experimental import pallas as pl


def kernel(x, edge_index, W1, b1, W2, b2, W3, b3, Wlin, blin):
    raise NotImplementedError("write your pallas kernel here")


# speedup vs baseline: 11.7775x; 11.7775x over previous
import functools

import jax
import jax.numpy as jnp
from jax import lax
from jax.experimental import pallas as pl
from jax.experimental.pallas import tpu as pltpu
from jax.experimental.pallas import tpu_sc as plsc

_N = 100000
_E = 3200000
_DIN = 10
_DH = 20
_NC = 4

_DS = 16
_NL = 100352

_NW = 32
_LANE = 128
_NCHUNK = 784
_EPAD = _NW * _NCHUNK * _LANE
_OUTR = _NL // 16

_BR = 3584
_GRID = _NL // _BR


def _sc_degree(dstp):
  mesh = plsc.VectorSubcoreMesh(core_axis_name="c", subcore_axis_name="s")

  @functools.partial(
      pl.kernel,
      out_type=jax.ShapeDtypeStruct((2 * _NL,), jnp.float32),
      mesh=mesh,
      scratch_types=[
          pltpu.VMEM((_LANE,), jnp.int32),
          pltpu.VMEM((_LANE,), jnp.float32),
          pltpu.VMEM((784,), jnp.float32),
          pltpu.VMEM_SHARED((_NL,), jnp.float32),
      ],
      compiler_params=pltpu.CompilerParams(use_tc_tiling_on_sc=False),
  )
  def k(dst_hbm, out_hbm, didx, ones_v, obuf, acc):
    cid = lax.axis_index("c")
    sid = lax.axis_index("s")
    w = sid * 2 + cid
    for i in range(_LANE // 16):
      ones_v[pl.ds(i * 16, 16)] = jnp.ones((16,), jnp.float32)

    def zstore(i, carry):
      obuf[pl.ds(i * 16, 16)] = jnp.zeros((16,), jnp.float32)
      return carry

    lax.fori_loop(0, 784 // 16, zstore, 0)
    zbase = sid * _OUTR

    def zloop(i, carry):
      pltpu.sync_copy(obuf, acc.at[pl.ds(zbase + i * 784, 784)])
      return carry

    lax.fori_loop(0, _OUTR // 784, zloop, 0)
    plsc.subcore_barrier()
    ebase = w * _NCHUNK * _LANE

    def chunk(cb, carry):
      pltpu.sync_copy(dst_hbm.at[pl.ds(ebase + cb * _LANE, _LANE)], didx)
      pltpu.sync_copy(ones_v, acc.at[didx], add=True)
      return carry

    lax.fori_loop(0, _NCHUNK, chunk, 0)
    plsc.subcore_barrier()
    ob = sid * _OUTR

    def oloop(i, carry):
      o = ob + i * 784
      pltpu.sync_copy(acc.at[pl.ds(o, 784)], obuf)
      pltpu.sync_copy(obuf, out_hbm.at[pl.ds(cid * _NL + o, 784)])
      return carry

    lax.fori_loop(0, _OUTR // 784, oloop, 0)

  return k(dstp)


def _sc_scatter(srcp, dstp, q):
  mesh = plsc.VectorSubcoreMesh(core_axis_name="c", subcore_axis_name="s")

  @functools.partial(
      pl.kernel,
      out_type=jax.ShapeDtypeStruct((2 * _NL, _DS), jnp.float32),
      mesh=mesh,
      scratch_types=[
          pltpu.VMEM((_LANE,), jnp.int32),
          pltpu.VMEM((_LANE,), jnp.int32),
          pltpu.VMEM((_LANE, _DS), jnp.float32),
          pltpu.VMEM_SHARED((_NL, _DS), jnp.float32),
          pltpu.SemaphoreType.DMA,
      ],
      compiler_params=pltpu.CompilerParams(use_tc_tiling_on_sc=False),
  )
  def k(src_hbm, dst_hbm, q_hbm, out_hbm, sidx, didx, rows, acc, sem):
    cid = lax.axis_index("c")
    sid = lax.axis_index("s")
    w = sid * 2 + cid

    def zstore(i, carry):
      rows[i, :] = jnp.zeros((_DS,), jnp.float32)
      return carry

    lax.fori_loop(0, _LANE, zstore, 0)
    zbase = sid * _OUTR

    def zloop(i, carry):
      pltpu.sync_copy(rows, acc.at[pl.ds(zbase + i * _LANE, _LANE)])
      return carry

    lax.fori_loop(0, _OUTR // _LANE, zloop, 0)
    plsc.subcore_barrier()
    ebase = w * _NCHUNK * _LANE

    def chunk(cb, carry):
      e0 = ebase + cb * _LANE
      pltpu.sync_copy(src_hbm.at[pl.ds(e0, _LANE)], sidx)
      pltpu.sync_copy(dst_hbm.at[pl.ds(e0, _LANE)], didx)
      pltpu.async_copy(q_hbm.at[sidx], rows, sem).wait()
      pltpu.sync_copy(rows, acc.at[didx], add=True)
      return carry

    lax.fori_loop(0, _NCHUNK, chunk, 0)
    plsc.subcore_barrier()
    ob = sid * _OUTR

    def oloop(i, carry):
      o = ob + i * _LANE
      pltpu.sync_copy(acc.at[pl.ds(o, _LANE)], rows)
      pltpu.sync_copy(rows, out_hbm.at[pl.ds(cid * _NL + o, _LANE)])
      return carry

    lax.fori_loop(0, _OUTR // _LANE, oloop, 0)

  return k(srcp, dstp, q)


def _row_mask(i, br):
  rows = i * br + lax.broadcasted_iota(jnp.int32, (br, 1), 0)
  return rows < _N


def _zero_pad_lanes(x, width):
  z = jnp.zeros((x.shape[0], width - x.shape[1]), jnp.float32)
  return jnp.concatenate([x, z], axis=1)


def _tc_pre(dis, x):
  def body(dis_ref, x_ref, q_ref):
    i = pl.program_id(0)
    q = jnp.where(_row_mask(i, _BR), dis_ref[...] * x_ref[...], 0.0)
    q_ref[...] = _zero_pad_lanes(q, _DS)

  return pl.pallas_call(
      body,
      grid=(_GRID,),
      in_specs=[
          pl.BlockSpec((_BR, 1), lambda i: (i, 0)),
          pl.BlockSpec((_BR, _DIN), lambda i: (i, 0)),
      ],
      out_specs=pl.BlockSpec((_BR, _DS), lambda i: (i, 0)),
      out_shape=jax.ShapeDtypeStruct((_NL, _DS), jnp.float32),
  )(dis, x)


def _layer_tail(t, dis, b, wn):
  pre = dis * jnp.dot(t, wn, preferred_element_type=jnp.float32) + b
  h = jnp.maximum(pre, 0.0)
  nrm = jnp.sqrt(jnp.sum(h * h, axis=1, keepdims=True))
  return h / jnp.maximum(nrm, 1e-12)


def _split_q(h, dis, i):
  qn = jnp.where(_row_mask(i, _BR), dis * h, 0.0)
  return qn[:, :_DS], _zero_pad_lanes(qn[:, _DS:], _DS)


def _tc_mid1(s_part, q, dis, b, w1):
  def body(s_ref, q_ref, dis_ref, b_ref, w_ref, h_ref, qa_ref, qb_ref):
    s2 = s_ref[...]
    t = (s2[0] + s2[1] + q_ref[...])[:, :_DIN]
    dis_b = dis_ref[...]
    h = _layer_tail(t, dis_b, b_ref[...], w_ref[...])
    h_ref[...] = h
    qa, qb = _split_q(h, dis_b, pl.program_id(0))
    qa_ref[...] = qa
    qb_ref[...] = qb

  return pl.pallas_call(
      body,
      grid=(_GRID,),
      in_specs=[
          pl.BlockSpec((2, _BR, _DS), lambda i: (0, i, 0)),
          pl.BlockSpec((_BR, _DS), lambda i: (i, 0)),
          pl.BlockSpec((_BR, 1), lambda i: (i, 0)),
          pl.BlockSpec((1, _DH), lambda i: (0, 0)),
          pl.BlockSpec((_DIN, _DH), lambda i: (0, 0)),
      ],
      out_specs=[
          pl.BlockSpec((_BR, _DH), lambda i: (i, 0)),
          pl.BlockSpec((_BR, _DS), lambda i: (i, 0)),
          pl.BlockSpec((_BR, _DS), lambda i: (i, 0)),
      ],
      out_shape=[
          jax.ShapeDtypeStruct((_NL, _DH), jnp.float32),
          jax.ShapeDtypeStruct((_NL, _DS), jnp.float32),
          jax.ShapeDtypeStruct((_NL, _DS), jnp.float32),
      ],
  )(s_part, q, dis, b, w1)


def _assemble_t(sa, sb, qa, qb):
  ta = sa[0] + sa[1] + qa
  tb = sb[0] + sb[1] + qb
  return jnp.concatenate([ta, tb[:, : _DH - _DS]], axis=1)


def _tc_mid2(sa_p, sb_p, qa, qb, dis, b, wn):
  def body(sa_ref, sb_ref, qa_ref, qb_ref, dis_ref, b_ref, w_ref,
           h_ref, qa2_ref, qb2_ref):
    t = _assemble_t(sa_ref[...], sb_ref[...], qa_ref[...], qb_ref[...])
    dis_b = dis_ref[...]
    h = _layer_tail(t, dis_b, b_ref[...], w_ref[...])
    h_ref[...] = h
    qa2, qb2 = _split_q(h, dis_b, pl.program_id(0))
    qa2_ref[...] = qa2
    qb2_ref[...] = qb2

  return pl.pallas_call(
      body,
      grid=(_GRID,),
      in_specs=[
          pl.BlockSpec((2, _BR, _DS), lambda i: (0, i, 0)),
          pl.BlockSpec((2, _BR, _DS), lambda i: (0, i, 0)),
          pl.BlockSpec((_BR, _DS), lambda i: (i, 0)),
          pl.BlockSpec((_BR, _DS), lambda i: (i, 0)),
          pl.BlockSpec((_BR, 1), lambda i: (i, 0)),
          pl.BlockSpec((1, _DH), lambda i: (0, 0)),
          pl.BlockSpec((_DH, _DH), lambda i: (0, 0)),
      ],
      out_specs=[
          pl.BlockSpec((_BR, _DH), lambda i: (i, 0)),
          pl.BlockSpec((_BR, _DS), lambda i: (i, 0)),
          pl.BlockSpec((_BR, _DS), lambda i: (i, 0)),
      ],
      out_shape=[
          jax.ShapeDtypeStruct((_NL, _DH), jnp.float32),
          jax.ShapeDtypeStruct((_NL, _DS), jnp.float32),
          jax.ShapeDtypeStruct((_NL, _DS), jnp.float32),
      ],
  )(sa_p, sb_p, qa, qb, dis, b, wn)


def _tc_fin(sa_p, sb_p, qa, qb, dis, b, w3, h1, h2, wl1, wl2, wl3, blin):
  def body(sa_ref, sb_ref, qa_ref, qb_ref, dis_ref, b_ref, w3_ref,
           h1_ref, h2_ref, wl1_ref, wl2_ref, wl3_ref, bl_ref, o_ref):
    t = _assemble_t(sa_ref[...], sb_ref[...], qa_ref[...], qb_ref[...])
    h3 = _layer_tail(t, dis_ref[...], b_ref[...], w3_ref[...])
    o = (
        jnp.dot(h1_ref[...], wl1_ref[...], preferred_element_type=jnp.float32)
        + jnp.dot(h2_ref[...], wl2_ref[...],
                  preferred_element_type=jnp.float32)
        + jnp.dot(h3, wl3_ref[...], preferred_element_type=jnp.float32)
        + bl_ref[...]
    )
    z = o - jnp.max(o, axis=1, keepdims=True)
    o_ref[...] = z - jnp.log(jnp.sum(jnp.exp(z), axis=1, keepdims=True))

  return pl.pallas_call(
      body,
      grid=(_GRID,),
      in_specs=[
          pl.BlockSpec((2, _BR, _DS), lambda i: (0, i, 0)),
          pl.BlockSpec((2, _BR, _DS), lambda i: (0, i, 0)),
          pl.BlockSpec((_BR, _DS), lambda i: (i, 0)),
          pl.BlockSpec((_BR, _DS), lambda i: (i, 0)),
          pl.BlockSpec((_BR, 1), lambda i: (i, 0)),
          pl.BlockSpec((1, _DH), lambda i: (0, 0)),
          pl.BlockSpec((_DH, _DH), lambda i: (0, 0)),
          pl.BlockSpec((_BR, _DH), lambda i: (i, 0)),
          pl.BlockSpec((_BR, _DH), lambda i: (i, 0)),
          pl.BlockSpec((_DH, _NC), lambda i: (0, 0)),
          pl.BlockSpec((_DH, _NC), lambda i: (0, 0)),
          pl.BlockSpec((_DH, _NC), lambda i: (0, 0)),
          pl.BlockSpec((1, _NC), lambda i: (0, 0)),
      ],
      out_specs=pl.BlockSpec((_BR, _NC), lambda i: (i, 0)),
      out_shape=jax.ShapeDtypeStruct((_N, _NC), jnp.float32),
  )(sa_p, sb_p, qa, qb, dis, b, w3, h1, h2, wl1, wl2, wl3, blin)


@jax.jit
def kernel(x, edge_index, W1, b1, W2, b2, W3, b3, Wlin, blin):
  e = edge_index.shape[1]
  pad = _EPAD - e
  fill = jnp.full((pad,), _N, jnp.int32)
  srcp = jnp.concatenate([edge_index[0], fill])
  dstp = jnp.concatenate([edge_index[1], fill])

  degp = _sc_degree(dstp)
  dis = lax.rsqrt(degp[:_NL] + degp[_NL:] + 1.0).reshape(_NL, 1)

  q1 = _tc_pre(dis, x)

  b1r = b1.reshape(1, _DH)
  b2r = b2.reshape(1, _DH)
  b3r = b3.reshape(1, _DH)

  s1 = _sc_scatter(srcp, dstp, q1).reshape(2, _NL, _DS)
  h1, q2a, q2b = _tc_mid1(s1, q1, dis, b1r, W1)

  s2a = _sc_scatter(srcp, dstp, q2a).reshape(2, _NL, _DS)
  s2b = _sc_scatter(srcp, dstp, q2b).reshape(2, _NL, _DS)
  h2, q3a, q3b = _tc_mid2(s2a, s2b, q2a, q2b, dis, b2r, W2)

  s3a = _sc_scatter(srcp, dstp, q3a).reshape(2, _NL, _DS)
  s3b = _sc_scatter(srcp, dstp, q3b).reshape(2, _NL, _DS)
  return _tc_fin(
      s3a, s3b, q3a, q3b, dis, b3r, W3, h1, h2,
      Wlin[:_DH], Wlin[_DH:2 * _DH], Wlin[2 * _DH:], blin.reshape(1, _NC),
  )


# speedup vs baseline: 17.6856x; 1.5016x over previous
import functools

import jax
import jax.numpy as jnp
from jax import lax
from jax.experimental import pallas as pl
from jax.experimental.pallas import tpu as pltpu
from jax.experimental.pallas import tpu_sc as plsc

_N = 100000
_E = 3200000
_DIN = 10
_DH = 20
_NC = 4

_DS = 16
_NL = 100352

_NW = 32
_LANE = 128
_NCHUNK = 784
_EPAD = _NW * _NCHUNK * _LANE
_OUTR = _NL // 16

_NBUF = 4

_BR = 3584
_GRID = _NL // _BR


def _sc_degree(dstp):
  mesh = plsc.VectorSubcoreMesh(core_axis_name="c", subcore_axis_name="s")

  @functools.partial(
      pl.kernel,
      out_type=jax.ShapeDtypeStruct((2 * _NL,), jnp.float32),
      mesh=mesh,
      scratch_types=[
          pltpu.VMEM((_LANE,), jnp.int32),
          pltpu.VMEM((_LANE,), jnp.float32),
          pltpu.VMEM((784,), jnp.float32),
          pltpu.VMEM_SHARED((_NL,), jnp.float32),
      ],
      compiler_params=pltpu.CompilerParams(use_tc_tiling_on_sc=False),
  )
  def k(dst_hbm, out_hbm, didx, ones_v, obuf, acc):
    cid = lax.axis_index("c")
    sid = lax.axis_index("s")
    w = sid * 2 + cid
    for i in range(_LANE // 16):
      ones_v[pl.ds(i * 16, 16)] = jnp.ones((16,), jnp.float32)

    def zstore(i, carry):
      obuf[pl.ds(i * 16, 16)] = jnp.zeros((16,), jnp.float32)
      return carry

    lax.fori_loop(0, 784 // 16, zstore, 0)
    zbase = sid * _OUTR

    def zloop(i, carry):
      pltpu.sync_copy(obuf, acc.at[pl.ds(zbase + i * 784, 784)])
      return carry

    lax.fori_loop(0, _OUTR // 784, zloop, 0)
    plsc.subcore_barrier()
    ebase = w * _NCHUNK * _LANE

    def chunk(cb, carry):
      pltpu.sync_copy(dst_hbm.at[pl.ds(ebase + cb * _LANE, _LANE)], didx)
      pltpu.sync_copy(ones_v, acc.at[didx], add=True)
      return carry

    lax.fori_loop(0, _NCHUNK, chunk, 0)
    plsc.subcore_barrier()
    ob = sid * _OUTR

    def oloop(i, carry):
      o = ob + i * 784
      pltpu.sync_copy(acc.at[pl.ds(o, 784)], obuf)
      pltpu.sync_copy(obuf, out_hbm.at[pl.ds(cid * _NL + o, 784)])
      return carry

    lax.fori_loop(0, _OUTR // 784, oloop, 0)

  return k(dstp)


def _sc_scatter(srcp, dstp, q):
  mesh = plsc.VectorSubcoreMesh(core_axis_name="c", subcore_axis_name="s")

  @functools.partial(
      pl.kernel,
      out_type=jax.ShapeDtypeStruct((2 * _NL, _DS), jnp.float32),
      mesh=mesh,
      scratch_types=[
          [pltpu.VMEM((_LANE,), jnp.int32) for _ in range(_NBUF)],
          [pltpu.VMEM((_LANE,), jnp.int32) for _ in range(_NBUF)],
          [pltpu.VMEM((_LANE, _DS), jnp.float32) for _ in range(_NBUF)],
          pltpu.VMEM_SHARED((_NL, _DS), jnp.float32),
          [pltpu.SemaphoreType.DMA for _ in range(_NBUF)],
      ],
      compiler_params=pltpu.CompilerParams(use_tc_tiling_on_sc=False),
  )
  def k(src_hbm, dst_hbm, q_hbm, out_hbm, sidx, didx, rows, acc, sem):
    cid = lax.axis_index("c")
    sid = lax.axis_index("s")
    w = sid * 2 + cid

    def zstore(i, carry):
      rows[0][i, :] = jnp.zeros((_DS,), jnp.float32)
      return carry

    lax.fori_loop(0, _LANE, zstore, 0)
    zbase = sid * _OUTR

    def zloop(i, carry):
      pltpu.sync_copy(rows[0], acc.at[pl.ds(zbase + i * _LANE, _LANE)])
      return carry

    lax.fori_loop(0, _OUTR // _LANE, zloop, 0)
    plsc.subcore_barrier()
    ebase = w * _NCHUNK * _LANE

    def start(cb, b):
      e0 = ebase + cb * _LANE
      pltpu.sync_copy(src_hbm.at[pl.ds(e0, _LANE)], sidx[b])
      pltpu.sync_copy(dst_hbm.at[pl.ds(e0, _LANE)], didx[b])
      pltpu.async_copy(q_hbm.at[sidx[b]], rows[b], sem[b])

    for b in range(_NBUF):
      start(b, b)

    def group(i, carry):
      for b in range(_NBUF):
        cb = i * _NBUF + b
        pltpu.make_async_copy(q_hbm.at[sidx[b]], rows[b], sem[b]).wait()
        pltpu.sync_copy(rows[b], acc.at[didx[b]], add=True)

        @pl.when(cb + _NBUF < _NCHUNK)
        def _():
          start(cb + _NBUF, b)

      return carry

    lax.fori_loop(0, _NCHUNK // _NBUF, group, 0)
    plsc.subcore_barrier()
    ob = sid * _OUTR

    def oloop(i, carry):
      o = ob + i * _LANE
      pltpu.sync_copy(acc.at[pl.ds(o, _LANE)], rows[0])
      pltpu.sync_copy(rows[0], out_hbm.at[pl.ds(cid * _NL + o, _LANE)])
      return carry

    lax.fori_loop(0, _OUTR // _LANE, oloop, 0)

  return k(srcp, dstp, q)


def _row_mask(i, br):
  rows = i * br + lax.broadcasted_iota(jnp.int32, (br, 1), 0)
  return rows < _N


def _zero_pad_lanes(x, width):
  z = jnp.zeros((x.shape[0], width - x.shape[1]), jnp.float32)
  return jnp.concatenate([x, z], axis=1)


def _tc_pre(dis, x):
  def body(dis_ref, x_ref, q_ref):
    i = pl.program_id(0)
    q = jnp.where(_row_mask(i, _BR), dis_ref[...] * x_ref[...], 0.0)
    q_ref[...] = _zero_pad_lanes(q, _DS)

  return pl.pallas_call(
      body,
      grid=(_GRID,),
      in_specs=[
          pl.BlockSpec((_BR, 1), lambda i: (i, 0)),
          pl.BlockSpec((_BR, _DIN), lambda i: (i, 0)),
      ],
      out_specs=pl.BlockSpec((_BR, _DS), lambda i: (i, 0)),
      out_shape=jax.ShapeDtypeStruct((_NL, _DS), jnp.float32),
  )(dis, x)


def _layer_tail(t, dis, b, wn):
  pre = dis * jnp.dot(t, wn, preferred_element_type=jnp.float32) + b
  h = jnp.maximum(pre, 0.0)
  nrm = jnp.sqrt(jnp.sum(h * h, axis=1, keepdims=True))
  return h / jnp.maximum(nrm, 1e-12)


def _split_q(h, dis, i):
  qn = jnp.where(_row_mask(i, _BR), dis * h, 0.0)
  return qn[:, :_DS], _zero_pad_lanes(qn[:, _DS:], _DS)


def _tc_mid1(s_part, q, dis, b, w1):
  def body(s_ref, q_ref, dis_ref, b_ref, w_ref, h_ref, qa_ref, qb_ref):
    s2 = s_ref[...]
    t = (s2[0] + s2[1] + q_ref[...])[:, :_DIN]
    dis_b = dis_ref[...]
    h = _layer_tail(t, dis_b, b_ref[...], w_ref[...])
    h_ref[...] = h
    qa, qb = _split_q(h, dis_b, pl.program_id(0))
    qa_ref[...] = qa
    qb_ref[...] = qb

  return pl.pallas_call(
      body,
      grid=(_GRID,),
      in_specs=[
          pl.BlockSpec((2, _BR, _DS), lambda i: (0, i, 0)),
          pl.BlockSpec((_BR, _DS), lambda i: (i, 0)),
          pl.BlockSpec((_BR, 1), lambda i: (i, 0)),
          pl.BlockSpec((1, _DH), lambda i: (0, 0)),
          pl.BlockSpec((_DIN, _DH), lambda i: (0, 0)),
      ],
      out_specs=[
          pl.BlockSpec((_BR, _DH), lambda i: (i, 0)),
          pl.BlockSpec((_BR, _DS), lambda i: (i, 0)),
          pl.BlockSpec((_BR, _DS), lambda i: (i, 0)),
      ],
      out_shape=[
          jax.ShapeDtypeStruct((_NL, _DH), jnp.float32),
          jax.ShapeDtypeStruct((_NL, _DS), jnp.float32),
          jax.ShapeDtypeStruct((_NL, _DS), jnp.float32),
      ],
  )(s_part, q, dis, b, w1)


def _assemble_t(sa, sb, qa, qb):
  ta = sa[0] + sa[1] + qa
  tb = sb[0] + sb[1] + qb
  return jnp.concatenate([ta, tb[:, : _DH - _DS]], axis=1)


def _tc_mid2(sa_p, sb_p, qa, qb, dis, b, wn):
  def body(sa_ref, sb_ref, qa_ref, qb_ref, dis_ref, b_ref, w_ref,
           h_ref, qa2_ref, qb2_ref):
    t = _assemble_t(sa_ref[...], sb_ref[...], qa_ref[...], qb_ref[...])
    dis_b = dis_ref[...]
    h = _layer_tail(t, dis_b, b_ref[...], w_ref[...])
    h_ref[...] = h
    qa2, qb2 = _split_q(h, dis_b, pl.program_id(0))
    qa2_ref[...] = qa2
    qb2_ref[...] = qb2

  return pl.pallas_call(
      body,
      grid=(_GRID,),
      in_specs=[
          pl.BlockSpec((2, _BR, _DS), lambda i: (0, i, 0)),
          pl.BlockSpec((2, _BR, _DS), lambda i: (0, i, 0)),
          pl.BlockSpec((_BR, _DS), lambda i: (i, 0)),
          pl.BlockSpec((_BR, _DS), lambda i: (i, 0)),
          pl.BlockSpec((_BR, 1), lambda i: (i, 0)),
          pl.BlockSpec((1, _DH), lambda i: (0, 0)),
          pl.BlockSpec((_DH, _DH), lambda i: (0, 0)),
      ],
      out_specs=[
          pl.BlockSpec((_BR, _DH), lambda i: (i, 0)),
          pl.BlockSpec((_BR, _DS), lambda i: (i, 0)),
          pl.BlockSpec((_BR, _DS), lambda i: (i, 0)),
      ],
      out_shape=[
          jax.ShapeDtypeStruct((_NL, _DH), jnp.float32),
          jax.ShapeDtypeStruct((_NL, _DS), jnp.float32),
          jax.ShapeDtypeStruct((_NL, _DS), jnp.float32),
      ],
  )(sa_p, sb_p, qa, qb, dis, b, wn)


def _tc_fin(sa_p, sb_p, qa, qb, dis, b, w3, h1, h2, wl1, wl2, wl3, blin):
  def body(sa_ref, sb_ref, qa_ref, qb_ref, dis_ref, b_ref, w3_ref,
           h1_ref, h2_ref, wl1_ref, wl2_ref, wl3_ref, bl_ref, o_ref):
    t = _assemble_t(sa_ref[...], sb_ref[...], qa_ref[...], qb_ref[...])
    h3 = _layer_tail(t, dis_ref[...], b_ref[...], w3_ref[...])
    o = (
        jnp.dot(h1_ref[...], wl1_ref[...], preferred_element_type=jnp.float32)
        + jnp.dot(h2_ref[...], wl2_ref[...],
                  preferred_element_type=jnp.float32)
        + jnp.dot(h3, wl3_ref[...], preferred_element_type=jnp.float32)
        + bl_ref[...]
    )
    z = o - jnp.max(o, axis=1, keepdims=True)
    o_ref[...] = z - jnp.log(jnp.sum(jnp.exp(z), axis=1, keepdims=True))

  return pl.pallas_call(
      body,
      grid=(_GRID,),
      in_specs=[
          pl.BlockSpec((2, _BR, _DS), lambda i: (0, i, 0)),
          pl.BlockSpec((2, _BR, _DS), lambda i: (0, i, 0)),
          pl.BlockSpec((_BR, _DS), lambda i: (i, 0)),
          pl.BlockSpec((_BR, _DS), lambda i: (i, 0)),
          pl.BlockSpec((_BR, 1), lambda i: (i, 0)),
          pl.BlockSpec((1, _DH), lambda i: (0, 0)),
          pl.BlockSpec((_DH, _DH), lambda i: (0, 0)),
          pl.BlockSpec((_BR, _DH), lambda i: (i, 0)),
          pl.BlockSpec((_BR, _DH), lambda i: (i, 0)),
          pl.BlockSpec((_DH, _NC), lambda i: (0, 0)),
          pl.BlockSpec((_DH, _NC), lambda i: (0, 0)),
          pl.BlockSpec((_DH, _NC), lambda i: (0, 0)),
          pl.BlockSpec((1, _NC), lambda i: (0, 0)),
      ],
      out_specs=pl.BlockSpec((_BR, _NC), lambda i: (i, 0)),
      out_shape=jax.ShapeDtypeStruct((_N, _NC), jnp.float32),
  )(sa_p, sb_p, qa, qb, dis, b, w3, h1, h2, wl1, wl2, wl3, blin)


@jax.jit
def kernel(x, edge_index, W1, b1, W2, b2, W3, b3, Wlin, blin):
  e = edge_index.shape[1]
  pad = _EPAD - e
  fill = jnp.full((pad,), _N, jnp.int32)
  srcp = jnp.concatenate([edge_index[0], fill])
  dstp = jnp.concatenate([edge_index[1], fill])

  degp = _sc_degree(dstp)
  dis = lax.rsqrt(degp[:_NL] + degp[_NL:] + 1.0).reshape(_NL, 1)

  q1 = _tc_pre(dis, x)

  b1r = b1.reshape(1, _DH)
  b2r = b2.reshape(1, _DH)
  b3r = b3.reshape(1, _DH)

  s1 = _sc_scatter(srcp, dstp, q1).reshape(2, _NL, _DS)
  h1, q2a, q2b = _tc_mid1(s1, q1, dis, b1r, W1)

  s2a = _sc_scatter(srcp, dstp, q2a).reshape(2, _NL, _DS)
  s2b = _sc_scatter(srcp, dstp, q2b).reshape(2, _NL, _DS)
  h2, q3a, q3b = _tc_mid2(s2a, s2b, q2a, q2b, dis, b2r, W2)

  s3a = _sc_scatter(srcp, dstp, q3a).reshape(2, _NL, _DS)
  s3b = _sc_scatter(srcp, dstp, q3b).reshape(2, _NL, _DS)
  return _tc_fin(
      s3a, s3b, q3a, q3b, dis, b3r, W3, h1, h2,
      Wlin[:_DH], Wlin[_DH:2 * _DH], Wlin[2 * _DH:], blin.reshape(1, _NC),
  )


# speedup vs baseline: 34.1433x; 1.9306x over previous
import functools

import jax
import jax.numpy as jnp
from jax import lax
from jax.experimental import pallas as pl
from jax.experimental.pallas import tpu as pltpu
from jax.experimental.pallas import tpu_sc as plsc

_N = 100000
_E = 3200000
_DIN = 10
_DH = 20
_NC = 4

_DS = 16
_NL = 100352

_NW = 32
_LANE = 128
_NCHUNK = 784
_EPAD = _NW * _NCHUNK * _LANE
_OUTR = _NL // 16

_NBUF = 4
_SG = 8
_NSUP = _NCHUNK // _SG

_BR = 3584
_GRID = _NL // _BR


def _sc_degree(dstp):
  mesh = plsc.VectorSubcoreMesh(core_axis_name="c", subcore_axis_name="s")

  @functools.partial(
      pl.kernel,
      out_type=jax.ShapeDtypeStruct((2 * _NL,), jnp.float32),
      mesh=mesh,
      scratch_types=[
          pltpu.VMEM((_LANE,), jnp.int32),
          pltpu.VMEM((_LANE,), jnp.float32),
          pltpu.VMEM((784,), jnp.float32),
          pltpu.VMEM_SHARED((_NL,), jnp.float32),
      ],
      compiler_params=pltpu.CompilerParams(use_tc_tiling_on_sc=False),
  )
  def k(dst_hbm, out_hbm, didx, ones_v, obuf, acc):
    cid = lax.axis_index("c")
    sid = lax.axis_index("s")
    w = sid * 2 + cid
    for i in range(_LANE // 16):
      ones_v[pl.ds(i * 16, 16)] = jnp.ones((16,), jnp.float32)

    def zstore(i, carry):
      obuf[pl.ds(i * 16, 16)] = jnp.zeros((16,), jnp.float32)
      return carry

    lax.fori_loop(0, 784 // 16, zstore, 0)
    zbase = sid * _OUTR

    def zloop(i, carry):
      pltpu.sync_copy(obuf, acc.at[pl.ds(zbase + i * 784, 784)])
      return carry

    lax.fori_loop(0, _OUTR // 784, zloop, 0)
    plsc.subcore_barrier()
    ebase = w * _NCHUNK * _LANE

    def chunk(cb, carry):
      pltpu.sync_copy(dst_hbm.at[pl.ds(ebase + cb * _LANE, _LANE)], didx)
      pltpu.sync_copy(ones_v, acc.at[didx], add=True)
      return carry

    lax.fori_loop(0, _NCHUNK, chunk, 0)
    plsc.subcore_barrier()
    ob = sid * _OUTR

    def oloop(i, carry):
      o = ob + i * 784
      pltpu.sync_copy(acc.at[pl.ds(o, 784)], obuf)
      pltpu.sync_copy(obuf, out_hbm.at[pl.ds(cid * _NL + o, 784)])
      return carry

    lax.fori_loop(0, _OUTR // 784, oloop, 0)

  return k(dstp)


def _sc_scatter(srcp, dstp, q):
  mesh = plsc.VectorSubcoreMesh(core_axis_name="c", subcore_axis_name="s")

  @functools.partial(
      pl.kernel,
      out_type=jax.ShapeDtypeStruct((2 * _NL, _DS), jnp.float32),
      mesh=mesh,
      scratch_types=[
          [pltpu.VMEM((_SG, _LANE), jnp.int32) for _ in range(2)],
          [pltpu.VMEM((_SG, _LANE), jnp.int32) for _ in range(2)],
          [pltpu.VMEM((_LANE, _DS), jnp.float32) for _ in range(_NBUF)],
          pltpu.VMEM_SHARED((_NL, _DS), jnp.float32),
          [pltpu.SemaphoreType.DMA for _ in range(_NBUF)],
          [pltpu.SemaphoreType.DMA for _ in range(2)],
      ],
      compiler_params=pltpu.CompilerParams(use_tc_tiling_on_sc=False),
  )
  def k(src_hbm, dst_hbm, q_hbm, out_hbm, sidx2, didx2, rows, acc, sem,
        semi):
    cid = lax.axis_index("c")
    sid = lax.axis_index("s")
    w = sid * 2 + cid

    def zstore(i, carry):
      rows[0][i, :] = jnp.zeros((_DS,), jnp.float32)
      return carry

    lax.fori_loop(0, _LANE, zstore, 0)
    zbase = sid * _OUTR

    def zloop(i, carry):
      pltpu.sync_copy(rows[0], acc.at[pl.ds(zbase + i * _LANE, _LANE)])
      return carry

    lax.fori_loop(0, _OUTR // _LANE, zloop, 0)
    plsc.subcore_barrier()
    rbase = w * _NCHUNK

    def load_idx(s, p):
      r0 = rbase + s * _SG
      pltpu.async_copy(src_hbm.at[pl.ds(r0, _SG)], sidx2[p], semi[p])
      pltpu.async_copy(dst_hbm.at[pl.ds(r0, _SG)], didx2[p], semi[p])

    def do_super(s, p):
      pltpu.make_async_copy(src_hbm.at[pl.ds(0, _SG)], sidx2[p],
                            semi[p]).wait()
      pltpu.make_async_copy(src_hbm.at[pl.ds(0, _SG)], didx2[p],
                            semi[p]).wait()
      for j in range(_NBUF):
        pltpu.async_copy(q_hbm.at[sidx2[p].at[j]], rows[j], sem[j])
      for j in range(_SG):
        b = j % _NBUF
        pltpu.make_async_copy(q_hbm.at[sidx2[p].at[j]], rows[b],
                              sem[b]).wait()
        pltpu.sync_copy(rows[b], acc.at[didx2[p].at[j]], add=True)
        if j + _NBUF < _SG:
          pltpu.async_copy(q_hbm.at[sidx2[p].at[j + _NBUF]], rows[b],
                           sem[b])

      @pl.when(s + 2 < _NSUP)
      def _():
        load_idx(s + 2, p)

    load_idx(0, 0)
    load_idx(1, 1)

    def group(i, carry):
      do_super(2 * i, 0)
      do_super(2 * i + 1, 1)
      return carry

    lax.fori_loop(0, _NSUP // 2, group, 0)
    plsc.subcore_barrier()
    ob = sid * _OUTR

    def oloop(i, carry):
      o = ob + i * _LANE
      pltpu.sync_copy(acc.at[pl.ds(o, _LANE)], rows[0])
      pltpu.sync_copy(rows[0], out_hbm.at[pl.ds(cid * _NL + o, _LANE)])
      return carry

    lax.fori_loop(0, _OUTR // _LANE, oloop, 0)

  return k(srcp, dstp, q)


def _row_mask(i, br):
  rows = i * br + lax.broadcasted_iota(jnp.int32, (br, 1), 0)
  return rows < _N


def _zero_pad_lanes(x, width):
  z = jnp.zeros((x.shape[0], width - x.shape[1]), jnp.float32)
  return jnp.concatenate([x, z], axis=1)


def _tc_pre(dis, x):
  def body(dis_ref, x_ref, q_ref):
    i = pl.program_id(0)
    q = jnp.where(_row_mask(i, _BR), dis_ref[...] * x_ref[...], 0.0)
    q_ref[...] = _zero_pad_lanes(q, _DS)

  return pl.pallas_call(
      body,
      grid=(_GRID,),
      in_specs=[
          pl.BlockSpec((_BR, 1), lambda i: (i, 0)),
          pl.BlockSpec((_BR, _DIN), lambda i: (i, 0)),
      ],
      out_specs=pl.BlockSpec((_BR, _DS), lambda i: (i, 0)),
      out_shape=jax.ShapeDtypeStruct((_NL, _DS), jnp.float32),
  )(dis, x)


def _layer_tail(t, dis, b, wn):
  pre = dis * jnp.dot(t, wn, preferred_element_type=jnp.float32) + b
  h = jnp.maximum(pre, 0.0)
  nrm = jnp.sqrt(jnp.sum(h * h, axis=1, keepdims=True))
  return h / jnp.maximum(nrm, 1e-12)


def _split_q(h, dis, i):
  qn = jnp.where(_row_mask(i, _BR), dis * h, 0.0)
  return qn[:, :_DS], _zero_pad_lanes(qn[:, _DS:], _DS)


def _tc_mid1(s_part, q, dis, b, w1):
  def body(s_ref, q_ref, dis_ref, b_ref, w_ref, h_ref, qa_ref, qb_ref):
    s2 = s_ref[...]
    t = (s2[0] + s2[1] + q_ref[...])[:, :_DIN]
    dis_b = dis_ref[...]
    h = _layer_tail(t, dis_b, b_ref[...], w_ref[...])
    h_ref[...] = h
    qa, qb = _split_q(h, dis_b, pl.program_id(0))
    qa_ref[...] = qa
    qb_ref[...] = qb

  return pl.pallas_call(
      body,
      grid=(_GRID,),
      in_specs=[
          pl.BlockSpec((2, _BR, _DS), lambda i: (0, i, 0)),
          pl.BlockSpec((_BR, _DS), lambda i: (i, 0)),
          pl.BlockSpec((_BR, 1), lambda i: (i, 0)),
          pl.BlockSpec((1, _DH), lambda i: (0, 0)),
          pl.BlockSpec((_DIN, _DH), lambda i: (0, 0)),
      ],
      out_specs=[
          pl.BlockSpec((_BR, _DH), lambda i: (i, 0)),
          pl.BlockSpec((_BR, _DS), lambda i: (i, 0)),
          pl.BlockSpec((_BR, _DS), lambda i: (i, 0)),
      ],
      out_shape=[
          jax.ShapeDtypeStruct((_NL, _DH), jnp.float32),
          jax.ShapeDtypeStruct((_NL, _DS), jnp.float32),
          jax.ShapeDtypeStruct((_NL, _DS), jnp.float32),
      ],
  )(s_part, q, dis, b, w1)


def _assemble_t(sa, sb, qa, qb):
  ta = sa[0] + sa[1] + qa
  tb = sb[0] + sb[1] + qb
  return jnp.concatenate([ta, tb[:, : _DH - _DS]], axis=1)


def _tc_mid2(sa_p, sb_p, qa, qb, dis, b, wn):
  def body(sa_ref, sb_ref, qa_ref, qb_ref, dis_ref, b_ref, w_ref,
           h_ref, qa2_ref, qb2_ref):
    t = _assemble_t(sa_ref[...], sb_ref[...], qa_ref[...], qb_ref[...])
    dis_b = dis_ref[...]
    h = _layer_tail(t, dis_b, b_ref[...], w_ref[...])
    h_ref[...] = h
    qa2, qb2 = _split_q(h, dis_b, pl.program_id(0))
    qa2_ref[...] = qa2
    qb2_ref[...] = qb2

  return pl.pallas_call(
      body,
      grid=(_GRID,),
      in_specs=[
          pl.BlockSpec((2, _BR, _DS), lambda i: (0, i, 0)),
          pl.BlockSpec((2, _BR, _DS), lambda i: (0, i, 0)),
          pl.BlockSpec((_BR, _DS), lambda i: (i, 0)),
          pl.BlockSpec((_BR, _DS), lambda i: (i, 0)),
          pl.BlockSpec((_BR, 1), lambda i: (i, 0)),
          pl.BlockSpec((1, _DH), lambda i: (0, 0)),
          pl.BlockSpec((_DH, _DH), lambda i: (0, 0)),
      ],
      out_specs=[
          pl.BlockSpec((_BR, _DH), lambda i: (i, 0)),
          pl.BlockSpec((_BR, _DS), lambda i: (i, 0)),
          pl.BlockSpec((_BR, _DS), lambda i: (i, 0)),
      ],
      out_shape=[
          jax.ShapeDtypeStruct((_NL, _DH), jnp.float32),
          jax.ShapeDtypeStruct((_NL, _DS), jnp.float32),
          jax.ShapeDtypeStruct((_NL, _DS), jnp.float32),
      ],
  )(sa_p, sb_p, qa, qb, dis, b, wn)


def _tc_fin(sa_p, sb_p, qa, qb, dis, b, w3, h1, h2, wl1, wl2, wl3, blin):
  def body(sa_ref, sb_ref, qa_ref, qb_ref, dis_ref, b_ref, w3_ref,
           h1_ref, h2_ref, wl1_ref, wl2_ref, wl3_ref, bl_ref, o_ref):
    t = _assemble_t(sa_ref[...], sb_ref[...], qa_ref[...], qb_ref[...])
    h3 = _layer_tail(t, dis_ref[...], b_ref[...], w3_ref[...])
    o = (
        jnp.dot(h1_ref[...], wl1_ref[...], preferred_element_type=jnp.float32)
        + jnp.dot(h2_ref[...], wl2_ref[...],
                  preferred_element_type=jnp.float32)
        + jnp.dot(h3, wl3_ref[...], preferred_element_type=jnp.float32)
        + bl_ref[...]
    )
    z = o - jnp.max(o, axis=1, keepdims=True)
    o_ref[...] = z - jnp.log(jnp.sum(jnp.exp(z), axis=1, keepdims=True))

  return pl.pallas_call(
      body,
      grid=(_GRID,),
      in_specs=[
          pl.BlockSpec((2, _BR, _DS), lambda i: (0, i, 0)),
          pl.BlockSpec((2, _BR, _DS), lambda i: (0, i, 0)),
          pl.BlockSpec((_BR, _DS), lambda i: (i, 0)),
          pl.BlockSpec((_BR, _DS), lambda i: (i, 0)),
          pl.BlockSpec((_BR, 1), lambda i: (i, 0)),
          pl.BlockSpec((1, _DH), lambda i: (0, 0)),
          pl.BlockSpec((_DH, _DH), lambda i: (0, 0)),
          pl.BlockSpec((_BR, _DH), lambda i: (i, 0)),
          pl.BlockSpec((_BR, _DH), lambda i: (i, 0)),
          pl.BlockSpec((_DH, _NC), lambda i: (0, 0)),
          pl.BlockSpec((_DH, _NC), lambda i: (0, 0)),
          pl.BlockSpec((_DH, _NC), lambda i: (0, 0)),
          pl.BlockSpec((1, _NC), lambda i: (0, 0)),
      ],
      out_specs=pl.BlockSpec((_BR, _NC), lambda i: (i, 0)),
      out_shape=jax.ShapeDtypeStruct((_N, _NC), jnp.float32),
  )(sa_p, sb_p, qa, qb, dis, b, w3, h1, h2, wl1, wl2, wl3, blin)


@jax.jit
def kernel(x, edge_index, W1, b1, W2, b2, W3, b3, Wlin, blin):
  e = edge_index.shape[1]
  pad = _EPAD - e
  fill = jnp.full((pad,), _N, jnp.int32)
  src1 = jnp.concatenate([edge_index[0], fill])
  dst1 = jnp.concatenate([edge_index[1], fill])
  srcp = src1.reshape(-1, _LANE)
  dstp = dst1.reshape(-1, _LANE)

  degp = _sc_degree(dst1)
  dis = lax.rsqrt(degp[:_NL] + degp[_NL:] + 1.0).reshape(_NL, 1)

  q1 = _tc_pre(dis, x)

  b1r = b1.reshape(1, _DH)
  b2r = b2.reshape(1, _DH)
  b3r = b3.reshape(1, _DH)

  s1 = _sc_scatter(srcp, dstp, q1).reshape(2, _NL, _DS)
  h1, q2a, q2b = _tc_mid1(s1, q1, dis, b1r, W1)

  s2a = _sc_scatter(srcp, dstp, q2a).reshape(2, _NL, _DS)
  s2b = _sc_scatter(srcp, dstp, q2b).reshape(2, _NL, _DS)
  h2, q3a, q3b = _tc_mid2(s2a, s2b, q2a, q2b, dis, b2r, W2)

  s3a = _sc_scatter(srcp, dstp, q3a).reshape(2, _NL, _DS)
  s3b = _sc_scatter(srcp, dstp, q3b).reshape(2, _NL, _DS)
  return _tc_fin(
      s3a, s3b, q3a, q3b, dis, b3r, W3, h1, h2,
      Wlin[:_DH], Wlin[_DH:2 * _DH], Wlin[2 * _DH:], blin.reshape(1, _NC),
  )


# speedup vs baseline: 38.9272x; 1.1401x over previous
import functools

import jax
import jax.numpy as jnp
from jax import lax
from jax.experimental import pallas as pl
from jax.experimental.pallas import tpu as pltpu
from jax.experimental.pallas import tpu_sc as plsc

_N = 100000
_E = 3200000
_DIN = 10
_DH = 20
_NC = 4

_DS = 16
_NL = 100352

_NW = 32
_LANE = 128
_NCHUNK = 784
_EPAD = _NW * _NCHUNK * _LANE
_OUTR = _NL // 16

_NBUF = 4
_SG = 8
_NSUP = _NCHUNK // _SG

_BR = 3584
_GRID = _NL // _BR


def _sc_degree(dstp):
  mesh = plsc.VectorSubcoreMesh(core_axis_name="c", subcore_axis_name="s")

  @functools.partial(
      pl.kernel,
      out_type=jax.ShapeDtypeStruct((2 * _NL,), jnp.float32),
      mesh=mesh,
      scratch_types=[
          [pltpu.VMEM((_SG, _LANE), jnp.int32) for _ in range(2)],
          pltpu.VMEM((_LANE,), jnp.float32),
          pltpu.VMEM((784,), jnp.float32),
          pltpu.VMEM_SHARED((_NL,), jnp.float32),
          [pltpu.SemaphoreType.DMA for _ in range(2)],
      ],
      compiler_params=pltpu.CompilerParams(use_tc_tiling_on_sc=False),
  )
  def k(dst_hbm, out_hbm, didx2, ones_v, obuf, acc, semi):
    cid = lax.axis_index("c")
    sid = lax.axis_index("s")
    w = sid * 2 + cid
    for i in range(_LANE // 16):
      ones_v[pl.ds(i * 16, 16)] = jnp.ones((16,), jnp.float32)

    def zstore(i, carry):
      obuf[pl.ds(i * 16, 16)] = jnp.zeros((16,), jnp.float32)
      return carry

    lax.fori_loop(0, 784 // 16, zstore, 0)
    zbase = sid * _OUTR

    def zloop(i, carry):
      pltpu.sync_copy(obuf, acc.at[pl.ds(zbase + i * 784, 784)])
      return carry

    lax.fori_loop(0, _OUTR // 784, zloop, 0)
    plsc.subcore_barrier()
    rbase = w * _NCHUNK

    def load_idx(s, p):
      pltpu.async_copy(
          dst_hbm.at[pl.ds(rbase + s * _SG, _SG)], didx2[p], semi[p]
      )

    def do_super(s, p):
      pltpu.make_async_copy(dst_hbm.at[pl.ds(0, _SG)], didx2[p],
                            semi[p]).wait()
      for j in range(_SG):
        pltpu.sync_copy(ones_v, acc.at[didx2[p].at[j]], add=True)

      @pl.when(s + 2 < _NSUP)
      def _():
        load_idx(s + 2, p)

    load_idx(0, 0)
    load_idx(1, 1)

    def group(i, carry):
      do_super(2 * i, 0)
      do_super(2 * i + 1, 1)
      return carry

    lax.fori_loop(0, _NSUP // 2, group, 0)
    plsc.subcore_barrier()
    ob = sid * _OUTR

    def oloop(i, carry):
      o = ob + i * 784
      pltpu.sync_copy(acc.at[pl.ds(o, 784)], obuf)
      pltpu.sync_copy(obuf, out_hbm.at[pl.ds(cid * _NL + o, 784)])
      return carry

    lax.fori_loop(0, _OUTR // 784, oloop, 0)

  return k(dstp)


def _sc_scatter(srcp, dstp, q):
  mesh = plsc.VectorSubcoreMesh(core_axis_name="c", subcore_axis_name="s")

  @functools.partial(
      pl.kernel,
      out_type=jax.ShapeDtypeStruct((2 * _NL, _DS), jnp.float32),
      mesh=mesh,
      scratch_types=[
          [pltpu.VMEM((_SG, _LANE), jnp.int32) for _ in range(2)],
          [pltpu.VMEM((_SG, _LANE), jnp.int32) for _ in range(2)],
          [pltpu.VMEM((_LANE, _DS), jnp.float32) for _ in range(_NBUF)],
          pltpu.VMEM_SHARED((_NL, _DS), jnp.float32),
          [pltpu.SemaphoreType.DMA for _ in range(_NBUF)],
          [pltpu.SemaphoreType.DMA for _ in range(2)],
      ],
      compiler_params=pltpu.CompilerParams(use_tc_tiling_on_sc=False),
  )
  def k(src_hbm, dst_hbm, q_hbm, out_hbm, sidx2, didx2, rows, acc, sem,
        semi):
    cid = lax.axis_index("c")
    sid = lax.axis_index("s")
    w = sid * 2 + cid

    def zstore(i, carry):
      rows[0][i, :] = jnp.zeros((_DS,), jnp.float32)
      return carry

    lax.fori_loop(0, _LANE, zstore, 0)
    zbase = sid * _OUTR

    def zloop(i, carry):
      pltpu.sync_copy(rows[0], acc.at[pl.ds(zbase + i * _LANE, _LANE)])
      return carry

    lax.fori_loop(0, _OUTR // _LANE, zloop, 0)
    plsc.subcore_barrier()
    rbase = w * _NCHUNK

    def load_idx(s, p):
      r0 = rbase + s * _SG
      pltpu.async_copy(src_hbm.at[pl.ds(r0, _SG)], sidx2[p], semi[p])
      pltpu.async_copy(dst_hbm.at[pl.ds(r0, _SG)], didx2[p], semi[p])

    def do_super(s, p):
      pltpu.make_async_copy(src_hbm.at[pl.ds(0, _SG)], sidx2[p],
                            semi[p]).wait()
      pltpu.make_async_copy(src_hbm.at[pl.ds(0, _SG)], didx2[p],
                            semi[p]).wait()
      for j in range(_NBUF):
        pltpu.async_copy(q_hbm.at[sidx2[p].at[j]], rows[j], sem[j])
      for j in range(_SG):
        b = j % _NBUF
        pltpu.make_async_copy(q_hbm.at[sidx2[p].at[j]], rows[b],
                              sem[b]).wait()
        pltpu.sync_copy(rows[b], acc.at[didx2[p].at[j]], add=True)
        if j + _NBUF < _SG:
          pltpu.async_copy(q_hbm.at[sidx2[p].at[j + _NBUF]], rows[b],
                           sem[b])

      @pl.when(s + 2 < _NSUP)
      def _():
        load_idx(s + 2, p)

    load_idx(0, 0)
    load_idx(1, 1)

    def group(i, carry):
      do_super(2 * i, 0)
      do_super(2 * i + 1, 1)
      return carry

    lax.fori_loop(0, _NSUP // 2, group, 0)
    plsc.subcore_barrier()
    ob = sid * _OUTR

    def oloop(i, carry):
      o = ob + i * _LANE
      pltpu.sync_copy(acc.at[pl.ds(o, _LANE)], rows[0])
      pltpu.sync_copy(rows[0], out_hbm.at[pl.ds(cid * _NL + o, _LANE)])
      return carry

    lax.fori_loop(0, _OUTR // _LANE, oloop, 0)

  return k(srcp, dstp, q)


def _row_mask(i, br):
  rows = i * br + lax.broadcasted_iota(jnp.int32, (br, 1), 0)
  return rows < _N


def _zero_pad_lanes(x, width):
  z = jnp.zeros((x.shape[0], width - x.shape[1]), jnp.float32)
  return jnp.concatenate([x, z], axis=1)


def _tc_pre(dis, x):
  def body(dis_ref, x_ref, q_ref):
    i = pl.program_id(0)
    q = jnp.where(_row_mask(i, _BR), dis_ref[...] * x_ref[...], 0.0)
    q_ref[...] = _zero_pad_lanes(q, _DS)

  return pl.pallas_call(
      body,
      grid=(_GRID,),
      in_specs=[
          pl.BlockSpec((_BR, 1), lambda i: (i, 0)),
          pl.BlockSpec((_BR, _DIN), lambda i: (i, 0)),
      ],
      out_specs=pl.BlockSpec((_BR, _DS), lambda i: (i, 0)),
      out_shape=jax.ShapeDtypeStruct((_NL, _DS), jnp.float32),
  )(dis, x)


def _layer_tail(t, dis, b, wn):
  pre = dis * jnp.dot(t, wn, preferred_element_type=jnp.float32) + b
  h = jnp.maximum(pre, 0.0)
  nrm = jnp.sqrt(jnp.sum(h * h, axis=1, keepdims=True))
  return h / jnp.maximum(nrm, 1e-12)


def _split_q(h, dis, i):
  qn = jnp.where(_row_mask(i, _BR), dis * h, 0.0)
  return qn[:, :_DS], _zero_pad_lanes(qn[:, _DS:], _DS)


def _tc_mid1(s_part, q, dis, b, w1):
  def body(s_ref, q_ref, dis_ref, b_ref, w_ref, h_ref, qa_ref, qb_ref):
    s2 = s_ref[...]
    t = (s2[0] + s2[1] + q_ref[...])[:, :_DIN]
    dis_b = dis_ref[...]
    h = _layer_tail(t, dis_b, b_ref[...], w_ref[...])
    h_ref[...] = h
    qa, qb = _split_q(h, dis_b, pl.program_id(0))
    qa_ref[...] = qa
    qb_ref[...] = qb

  return pl.pallas_call(
      body,
      grid=(_GRID,),
      in_specs=[
          pl.BlockSpec((2, _BR, _DS), lambda i: (0, i, 0)),
          pl.BlockSpec((_BR, _DS), lambda i: (i, 0)),
          pl.BlockSpec((_BR, 1), lambda i: (i, 0)),
          pl.BlockSpec((1, _DH), lambda i: (0, 0)),
          pl.BlockSpec((_DIN, _DH), lambda i: (0, 0)),
      ],
      out_specs=[
          pl.BlockSpec((_BR, _DH), lambda i: (i, 0)),
          pl.BlockSpec((_BR, _DS), lambda i: (i, 0)),
          pl.BlockSpec((_BR, _DS), lambda i: (i, 0)),
      ],
      out_shape=[
          jax.ShapeDtypeStruct((_NL, _DH), jnp.float32),
          jax.ShapeDtypeStruct((_NL, _DS), jnp.float32),
          jax.ShapeDtypeStruct((_NL, _DS), jnp.float32),
      ],
  )(s_part, q, dis, b, w1)


def _assemble_t(sa, sb, qa, qb):
  ta = sa[0] + sa[1] + qa
  tb = sb[0] + sb[1] + qb
  return jnp.concatenate([ta, tb[:, : _DH - _DS]], axis=1)


def _tc_mid2(sa_p, sb_p, qa, qb, dis, b, wn):
  def body(sa_ref, sb_ref, qa_ref, qb_ref, dis_ref, b_ref, w_ref,
           h_ref, qa2_ref, qb2_ref):
    t = _assemble_t(sa_ref[...], sb_ref[...], qa_ref[...], qb_ref[...])
    dis_b = dis_ref[...]
    h = _layer_tail(t, dis_b, b_ref[...], w_ref[...])
    h_ref[...] = h
    qa2, qb2 = _split_q(h, dis_b, pl.program_id(0))
    qa2_ref[...] = qa2
    qb2_ref[...] = qb2

  return pl.pallas_call(
      body,
      grid=(_GRID,),
      in_specs=[
          pl.BlockSpec((2, _BR, _DS), lambda i: (0, i, 0)),
          pl.BlockSpec((2, _BR, _DS), lambda i: (0, i, 0)),
          pl.BlockSpec((_BR, _DS), lambda i: (i, 0)),
          pl.BlockSpec((_BR, _DS), lambda i: (i, 0)),
          pl.BlockSpec((_BR, 1), lambda i: (i, 0)),
          pl.BlockSpec((1, _DH), lambda i: (0, 0)),
          pl.BlockSpec((_DH, _DH), lambda i: (0, 0)),
      ],
      out_specs=[
          pl.BlockSpec((_BR, _DH), lambda i: (i, 0)),
          pl.BlockSpec((_BR, _DS), lambda i: (i, 0)),
          pl.BlockSpec((_BR, _DS), lambda i: (i, 0)),
      ],
      out_shape=[
          jax.ShapeDtypeStruct((_NL, _DH), jnp.float32),
          jax.ShapeDtypeStruct((_NL, _DS), jnp.float32),
          jax.ShapeDtypeStruct((_NL, _DS), jnp.float32),
      ],
  )(sa_p, sb_p, qa, qb, dis, b, wn)


def _tc_fin(sa_p, sb_p, qa, qb, dis, b, w3, h1, h2, wl1, wl2, wl3, blin):
  def body(sa_ref, sb_ref, qa_ref, qb_ref, dis_ref, b_ref, w3_ref,
           h1_ref, h2_ref, wl1_ref, wl2_ref, wl3_ref, bl_ref, o_ref):
    t = _assemble_t(sa_ref[...], sb_ref[...], qa_ref[...], qb_ref[...])
    h3 = _layer_tail(t, dis_ref[...], b_ref[...], w3_ref[...])
    o = (
        jnp.dot(h1_ref[...], wl1_ref[...], preferred_element_type=jnp.float32)
        + jnp.dot(h2_ref[...], wl2_ref[...],
                  preferred_element_type=jnp.float32)
        + jnp.dot(h3, wl3_ref[...], preferred_element_type=jnp.float32)
        + bl_ref[...]
    )
    z = o - jnp.max(o, axis=1, keepdims=True)
    o_ref[...] = z - jnp.log(jnp.sum(jnp.exp(z), axis=1, keepdims=True))

  return pl.pallas_call(
      body,
      grid=(_GRID,),
      in_specs=[
          pl.BlockSpec((2, _BR, _DS), lambda i: (0, i, 0)),
          pl.BlockSpec((2, _BR, _DS), lambda i: (0, i, 0)),
          pl.BlockSpec((_BR, _DS), lambda i: (i, 0)),
          pl.BlockSpec((_BR, _DS), lambda i: (i, 0)),
          pl.BlockSpec((_BR, 1), lambda i: (i, 0)),
          pl.BlockSpec((1, _DH), lambda i: (0, 0)),
          pl.BlockSpec((_DH, _DH), lambda i: (0, 0)),
          pl.BlockSpec((_BR, _DH), lambda i: (i, 0)),
          pl.BlockSpec((_BR, _DH), lambda i: (i, 0)),
          pl.BlockSpec((_DH, _NC), lambda i: (0, 0)),
          pl.BlockSpec((_DH, _NC), lambda i: (0, 0)),
          pl.BlockSpec((_DH, _NC), lambda i: (0, 0)),
          pl.BlockSpec((1, _NC), lambda i: (0, 0)),
      ],
      out_specs=pl.BlockSpec((_BR, _NC), lambda i: (i, 0)),
      out_shape=jax.ShapeDtypeStruct((_N, _NC), jnp.float32),
  )(sa_p, sb_p, qa, qb, dis, b, w3, h1, h2, wl1, wl2, wl3, blin)


@jax.jit
def kernel(x, edge_index, W1, b1, W2, b2, W3, b3, Wlin, blin):
  e = edge_index.shape[1]
  pad = _EPAD - e
  fill = jnp.full((pad,), _N, jnp.int32)
  srcp = jnp.concatenate([edge_index[0], fill]).reshape(-1, _LANE)
  dstp = jnp.concatenate([edge_index[1], fill]).reshape(-1, _LANE)

  degp = _sc_degree(dstp)
  dis = lax.rsqrt(degp[:_NL] + degp[_NL:] + 1.0).reshape(_NL, 1)

  q1 = _tc_pre(dis, x)

  b1r = b1.reshape(1, _DH)
  b2r = b2.reshape(1, _DH)
  b3r = b3.reshape(1, _DH)

  s1 = _sc_scatter(srcp, dstp, q1).reshape(2, _NL, _DS)
  h1, q2a, q2b = _tc_mid1(s1, q1, dis, b1r, W1)

  s2a = _sc_scatter(srcp, dstp, q2a).reshape(2, _NL, _DS)
  s2b = _sc_scatter(srcp, dstp, q2b).reshape(2, _NL, _DS)
  h2, q3a, q3b = _tc_mid2(s2a, s2b, q2a, q2b, dis, b2r, W2)

  s3a = _sc_scatter(srcp, dstp, q3a).reshape(2, _NL, _DS)
  s3b = _sc_scatter(srcp, dstp, q3b).reshape(2, _NL, _DS)
  return _tc_fin(
      s3a, s3b, q3a, q3b, dis, b3r, W3, h1, h2,
      Wlin[:_DH], Wlin[_DH:2 * _DH], Wlin[2 * _DH:], blin.reshape(1, _NC),
  )


# speedup vs baseline: 42.5304x; 1.0926x over previous
import functools

import jax
import jax.numpy as jnp
from jax import lax
from jax.experimental import pallas as pl
from jax.experimental.pallas import tpu as pltpu
from jax.experimental.pallas import tpu_sc as plsc

_N = 100000
_E = 3200000
_DIN = 10
_DH = 20
_NC = 4

_DS = 16
_NL = 100352

_NW = 32
_LANE = 128
_NCHUNK = 784
_EPAD = _NW * _NCHUNK * _LANE
_OUTR = _NL // 16

_NBUF = 4
_SG = 8
_NSUP = _NCHUNK // _SG

_BR = 3584
_GRID = _NL // _BR


def _sc_degree(dstp):
  mesh = plsc.VectorSubcoreMesh(core_axis_name="c", subcore_axis_name="s")

  @functools.partial(
      pl.kernel,
      out_type=jax.ShapeDtypeStruct((2 * _NL,), jnp.float32),
      mesh=mesh,
      scratch_types=[
          [pltpu.VMEM((_SG, _LANE), jnp.int32) for _ in range(2)],
          pltpu.VMEM((_LANE,), jnp.float32),
          pltpu.VMEM((784,), jnp.float32),
          pltpu.VMEM_SHARED((_NL,), jnp.float32),
          [pltpu.SemaphoreType.DMA for _ in range(2)],
      ],
      compiler_params=pltpu.CompilerParams(use_tc_tiling_on_sc=False),
  )
  def k(dst_hbm, out_hbm, didx2, ones_v, obuf, acc, semi):
    cid = lax.axis_index("c")
    sid = lax.axis_index("s")
    w = sid * 2 + cid
    for i in range(_LANE // 16):
      ones_v[pl.ds(i * 16, 16)] = jnp.ones((16,), jnp.float32)

    def zstore(i, carry):
      obuf[pl.ds(i * 16, 16)] = jnp.zeros((16,), jnp.float32)
      return carry

    lax.fori_loop(0, 784 // 16, zstore, 0)
    zbase = sid * _OUTR

    def zloop(i, carry):
      pltpu.sync_copy(obuf, acc.at[pl.ds(zbase + i * 784, 784)])
      return carry

    lax.fori_loop(0, _OUTR // 784, zloop, 0)
    plsc.subcore_barrier()
    rbase = w * _NCHUNK

    def load_idx(s, p):
      pltpu.async_copy(
          dst_hbm.at[pl.ds(rbase + s * _SG, _SG)], didx2[p], semi[p]
      )

    def do_super(s, p):
      pltpu.make_async_copy(dst_hbm.at[pl.ds(0, _SG)], didx2[p],
                            semi[p]).wait()
      for j in range(_SG):
        pltpu.sync_copy(ones_v, acc.at[didx2[p].at[j]], add=True)

      @pl.when(s + 2 < _NSUP)
      def _():
        load_idx(s + 2, p)

    load_idx(0, 0)
    load_idx(1, 1)

    def group(i, carry):
      do_super(2 * i, 0)
      do_super(2 * i + 1, 1)
      return carry

    lax.fori_loop(0, _NSUP // 2, group, 0)
    plsc.subcore_barrier()
    ob = sid * _OUTR

    def oloop(i, carry):
      o = ob + i * 784
      pltpu.sync_copy(acc.at[pl.ds(o, 784)], obuf)
      pltpu.sync_copy(obuf, out_hbm.at[pl.ds(cid * _NL + o, 784)])
      return carry

    lax.fori_loop(0, _OUTR // 784, oloop, 0)

  return k(dstp)


def _sc_scatter(srcp, dstp, q):
  mesh = plsc.VectorSubcoreMesh(core_axis_name="c", subcore_axis_name="s")

  @functools.partial(
      pl.kernel,
      out_type=jax.ShapeDtypeStruct((2 * _NL, _DS), jnp.float32),
      mesh=mesh,
      scratch_types=[
          [pltpu.VMEM((_SG, _LANE), jnp.int32) for _ in range(2)],
          [pltpu.VMEM((_SG, _LANE), jnp.int32) for _ in range(2)],
          [pltpu.VMEM((_LANE, _DS), jnp.float32) for _ in range(_SG)],
          pltpu.VMEM_SHARED((_NL, _DS), jnp.float32),
          [pltpu.SemaphoreType.DMA for _ in range(_SG)],
          [pltpu.SemaphoreType.DMA for _ in range(_SG)],
          [pltpu.SemaphoreType.DMA for _ in range(2)],
      ],
      compiler_params=pltpu.CompilerParams(use_tc_tiling_on_sc=False),
  )
  def k(src_hbm, dst_hbm, q_hbm, out_hbm, sidx2, didx2, rows, acc, sem,
        sems, semi):
    cid = lax.axis_index("c")
    sid = lax.axis_index("s")
    w = sid * 2 + cid

    def zstore(i, carry):
      rows[0][i, :] = jnp.zeros((_DS,), jnp.float32)
      return carry

    lax.fori_loop(0, _LANE, zstore, 0)
    zbase = sid * _OUTR

    def zloop(i, carry):
      pltpu.sync_copy(rows[0], acc.at[pl.ds(zbase + i * _LANE, _LANE)])
      return carry

    lax.fori_loop(0, _OUTR // _LANE, zloop, 0)
    plsc.subcore_barrier()
    rbase = w * _NCHUNK

    def load_idx(s, p):
      r0 = rbase + s * _SG
      pltpu.async_copy(src_hbm.at[pl.ds(r0, _SG)], sidx2[p], semi[p])
      pltpu.async_copy(dst_hbm.at[pl.ds(r0, _SG)], didx2[p], semi[p])

    def drain_scatters(p):
      for j in range(_SG):
        pltpu.make_async_copy(
            rows[j], acc.at[didx2[p].at[j]], sems[j]
        ).wait()

    def do_super(s, p):
      pltpu.make_async_copy(src_hbm.at[pl.ds(0, _SG)], sidx2[p],
                            semi[p]).wait()
      pltpu.make_async_copy(src_hbm.at[pl.ds(0, _SG)], didx2[p],
                            semi[p]).wait()
      @pl.when(s > 0)
      def _():
        drain_scatters(1 - p)

      @pl.when(s + 1 < _NSUP)
      def _():
        load_idx(s + 1, 1 - p)

      for j in range(_SG):
        pltpu.async_copy(q_hbm.at[sidx2[p].at[j]], rows[j], sem[j])
      for j in range(_SG):
        pltpu.make_async_copy(q_hbm.at[sidx2[p].at[j]], rows[j],
                              sem[j]).wait()
        pltpu.async_copy(rows[j], acc.at[didx2[p].at[j]], sems[j],
                         add=True)

    load_idx(0, 0)

    def group(i, carry):
      do_super(2 * i, 0)
      do_super(2 * i + 1, 1)
      return carry

    lax.fori_loop(0, _NSUP // 2, group, 0)
    drain_scatters(1)
    plsc.subcore_barrier()
    ob = sid * _OUTR

    def oloop(i, carry):
      o = ob + i * _LANE
      pltpu.sync_copy(acc.at[pl.ds(o, _LANE)], rows[0])
      pltpu.sync_copy(rows[0], out_hbm.at[pl.ds(cid * _NL + o, _LANE)])
      return carry

    lax.fori_loop(0, _OUTR // _LANE, oloop, 0)

  return k(srcp, dstp, q)


def _row_mask(i, br):
  rows = i * br + lax.broadcasted_iota(jnp.int32, (br, 1), 0)
  return rows < _N


def _zero_pad_lanes(x, width):
  z = jnp.zeros((x.shape[0], width - x.shape[1]), jnp.float32)
  return jnp.concatenate([x, z], axis=1)


def _tc_pre(dis, x):
  def body(dis_ref, x_ref, q_ref):
    i = pl.program_id(0)
    q = jnp.where(_row_mask(i, _BR), dis_ref[...] * x_ref[...], 0.0)
    q_ref[...] = _zero_pad_lanes(q, _DS)

  return pl.pallas_call(
      body,
      grid=(_GRID,),
      in_specs=[
          pl.BlockSpec((_BR, 1), lambda i: (i, 0)),
          pl.BlockSpec((_BR, _DIN), lambda i: (i, 0)),
      ],
      out_specs=pl.BlockSpec((_BR, _DS), lambda i: (i, 0)),
      out_shape=jax.ShapeDtypeStruct((_NL, _DS), jnp.float32),
  )(dis, x)


def _layer_tail(t, dis, b, wn):
  pre = dis * jnp.dot(t, wn, preferred_element_type=jnp.float32) + b
  h = jnp.maximum(pre, 0.0)
  nrm = jnp.sqrt(jnp.sum(h * h, axis=1, keepdims=True))
  return h / jnp.maximum(nrm, 1e-12)


def _split_q(h, dis, i):
  qn = jnp.where(_row_mask(i, _BR), dis * h, 0.0)
  return qn[:, :_DS], _zero_pad_lanes(qn[:, _DS:], _DS)


def _tc_mid1(s_part, q, dis, b, w1):
  def body(s_ref, q_ref, dis_ref, b_ref, w_ref, h_ref, qa_ref, qb_ref):
    s2 = s_ref[...]
    t = (s2[0] + s2[1] + q_ref[...])[:, :_DIN]
    dis_b = dis_ref[...]
    h = _layer_tail(t, dis_b, b_ref[...], w_ref[...])
    h_ref[...] = h
    qa, qb = _split_q(h, dis_b, pl.program_id(0))
    qa_ref[...] = qa
    qb_ref[...] = qb

  return pl.pallas_call(
      body,
      grid=(_GRID,),
      in_specs=[
          pl.BlockSpec((2, _BR, _DS), lambda i: (0, i, 0)),
          pl.BlockSpec((_BR, _DS), lambda i: (i, 0)),
          pl.BlockSpec((_BR, 1), lambda i: (i, 0)),
          pl.BlockSpec((1, _DH), lambda i: (0, 0)),
          pl.BlockSpec((_DIN, _DH), lambda i: (0, 0)),
      ],
      out_specs=[
          pl.BlockSpec((_BR, _DH), lambda i: (i, 0)),
          pl.BlockSpec((_BR, _DS), lambda i: (i, 0)),
          pl.BlockSpec((_BR, _DS), lambda i: (i, 0)),
      ],
      out_shape=[
          jax.ShapeDtypeStruct((_NL, _DH), jnp.float32),
          jax.ShapeDtypeStruct((_NL, _DS), jnp.float32),
          jax.ShapeDtypeStruct((_NL, _DS), jnp.float32),
      ],
  )(s_part, q, dis, b, w1)


def _assemble_t(sa, sb, qa, qb):
  ta = sa[0] + sa[1] + qa
  tb = sb[0] + sb[1] + qb
  return jnp.concatenate([ta, tb[:, : _DH - _DS]], axis=1)


def _tc_mid2(sa_p, sb_p, qa, qb, dis, b, wn):
  def body(sa_ref, sb_ref, qa_ref, qb_ref, dis_ref, b_ref, w_ref,
           h_ref, qa2_ref, qb2_ref):
    t = _assemble_t(sa_ref[...], sb_ref[...], qa_ref[...], qb_ref[...])
    dis_b = dis_ref[...]
    h = _layer_tail(t, dis_b, b_ref[...], w_ref[...])
    h_ref[...] = h
    qa2, qb2 = _split_q(h, dis_b, pl.program_id(0))
    qa2_ref[...] = qa2
    qb2_ref[...] = qb2

  return pl.pallas_call(
      body,
      grid=(_GRID,),
      in_specs=[
          pl.BlockSpec((2, _BR, _DS), lambda i: (0, i, 0)),
          pl.BlockSpec((2, _BR, _DS), lambda i: (0, i, 0)),
          pl.BlockSpec((_BR, _DS), lambda i: (i, 0)),
          pl.BlockSpec((_BR, _DS), lambda i: (i, 0)),
          pl.BlockSpec((_BR, 1), lambda i: (i, 0)),
          pl.BlockSpec((1, _DH), lambda i: (0, 0)),
          pl.BlockSpec((_DH, _DH), lambda i: (0, 0)),
      ],
      out_specs=[
          pl.BlockSpec((_BR, _DH), lambda i: (i, 0)),
          pl.BlockSpec((_BR, _DS), lambda i: (i, 0)),
          pl.BlockSpec((_BR, _DS), lambda i: (i, 0)),
      ],
      out_shape=[
          jax.ShapeDtypeStruct((_NL, _DH), jnp.float32),
          jax.ShapeDtypeStruct((_NL, _DS), jnp.float32),
          jax.ShapeDtypeStruct((_NL, _DS), jnp.float32),
      ],
  )(sa_p, sb_p, qa, qb, dis, b, wn)


def _tc_fin(sa_p, sb_p, qa, qb, dis, b, w3, h1, h2, wl1, wl2, wl3, blin):
  def body(sa_ref, sb_ref, qa_ref, qb_ref, dis_ref, b_ref, w3_ref,
           h1_ref, h2_ref, wl1_ref, wl2_ref, wl3_ref, bl_ref, o_ref):
    t = _assemble_t(sa_ref[...], sb_ref[...], qa_ref[...], qb_ref[...])
    h3 = _layer_tail(t, dis_ref[...], b_ref[...], w3_ref[...])
    o = (
        jnp.dot(h1_ref[...], wl1_ref[...], preferred_element_type=jnp.float32)
        + jnp.dot(h2_ref[...], wl2_ref[...],
                  preferred_element_type=jnp.float32)
        + jnp.dot(h3, wl3_ref[...], preferred_element_type=jnp.float32)
        + bl_ref[...]
    )
    z = o - jnp.max(o, axis=1, keepdims=True)
    o_ref[...] = z - jnp.log(jnp.sum(jnp.exp(z), axis=1, keepdims=True))

  return pl.pallas_call(
      body,
      grid=(_GRID,),
      in_specs=[
          pl.BlockSpec((2, _BR, _DS), lambda i: (0, i, 0)),
          pl.BlockSpec((2, _BR, _DS), lambda i: (0, i, 0)),
          pl.BlockSpec((_BR, _DS), lambda i: (i, 0)),
          pl.BlockSpec((_BR, _DS), lambda i: (i, 0)),
          pl.BlockSpec((_BR, 1), lambda i: (i, 0)),
          pl.BlockSpec((1, _DH), lambda i: (0, 0)),
          pl.BlockSpec((_DH, _DH), lambda i: (0, 0)),
          pl.BlockSpec((_BR, _DH), lambda i: (i, 0)),
          pl.BlockSpec((_BR, _DH), lambda i: (i, 0)),
          pl.BlockSpec((_DH, _NC), lambda i: (0, 0)),
          pl.BlockSpec((_DH, _NC), lambda i: (0, 0)),
          pl.BlockSpec((_DH, _NC), lambda i: (0, 0)),
          pl.BlockSpec((1, _NC), lambda i: (0, 0)),
      ],
      out_specs=pl.BlockSpec((_BR, _NC), lambda i: (i, 0)),
      out_shape=jax.ShapeDtypeStruct((_N, _NC), jnp.float32),
  )(sa_p, sb_p, qa, qb, dis, b, w3, h1, h2, wl1, wl2, wl3, blin)


@jax.jit
def kernel(x, edge_index, W1, b1, W2, b2, W3, b3, Wlin, blin):
  e = edge_index.shape[1]
  pad = _EPAD - e
  fill = jnp.full((pad,), _N, jnp.int32)
  srcp = jnp.concatenate([edge_index[0], fill]).reshape(-1, _LANE)
  dstp = jnp.concatenate([edge_index[1], fill]).reshape(-1, _LANE)

  degp = _sc_degree(dstp)
  dis = lax.rsqrt(degp[:_NL] + degp[_NL:] + 1.0).reshape(_NL, 1)

  q1 = _tc_pre(dis, x)

  b1r = b1.reshape(1, _DH)
  b2r = b2.reshape(1, _DH)
  b3r = b3.reshape(1, _DH)

  s1 = _sc_scatter(srcp, dstp, q1).reshape(2, _NL, _DS)
  h1, q2a, q2b = _tc_mid1(s1, q1, dis, b1r, W1)

  s2a = _sc_scatter(srcp, dstp, q2a).reshape(2, _NL, _DS)
  s2b = _sc_scatter(srcp, dstp, q2b).reshape(2, _NL, _DS)
  h2, q3a, q3b = _tc_mid2(s2a, s2b, q2a, q2b, dis, b2r, W2)

  s3a = _sc_scatter(srcp, dstp, q3a).reshape(2, _NL, _DS)
  s3b = _sc_scatter(srcp, dstp, q3b).reshape(2, _NL, _DS)
  return _tc_fin(
      s3a, s3b, q3a, q3b, dis, b3r, W3, h1, h2,
      Wlin[:_DH], Wlin[_DH:2 * _DH], Wlin[2 * _DH:], blin.reshape(1, _NC),
  )
